# baseline (speedup 1.0000x reference)
import functools

import jax
import jax.numpy as jnp
from jax import lax
from jax.experimental import pallas as pl
from jax.experimental.pallas import tpu as pltpu
from jax.experimental.pallas import tpu_sc as plsc

_THETAS = [
    [5.0, -10.0, 7.5, -2.5, 0.3125],
    [0.0, 10.0, -15.0, 7.5, -1.25],
    [0.0, 0.0, 7.5, -7.5, 1.875],
    [0.0, 0.0, 0.0, 2.5, -1.25],
    [0.0, 0.0, 0.0, 0.0, 0.3125],
]

_N = 10000
_DIM = 128
_E = 320000
_NC = 2
_NS = 16
_NW = _NC * _NS
_CH = 128
_CPT = 80
_EPAD = _NW * _CPT * _CH
_NPAD = 10112
_RPT = _NPAD // _NS

_sc_mesh = plsc.VectorSubcoreMesh(
    core_axis_name="c", subcore_axis_name="s", num_cores=_NC, num_subcores=_NS
)


_GRP = 16


def _agg_body(hw_hbm, src_hbm, dst_hbm, zero_hbm, out_hbm,
              src_v, dst_v, buf_a, buf_b, acc, semg_a, semg_b, sems_a, sems_b):
    c = lax.axis_index("c")
    s = lax.axis_index("s")
    w = s * _NC + c
    base = w * _CPT
    row0 = s * _RPT

    pltpu.sync_copy(zero_hbm.at[pl.ds(row0, _RPT)], acc.at[pl.ds(row0, _RPT)])
    plsc.subcore_barrier()

    def _wait_gather(buf, sem):
        pltpu.make_async_copy(hw_hbm.at[src_v.at[0]], buf, sem).wait()

    def _wait_scatter(buf, sem):
        pltpu.make_async_copy(buf, acc.at[dst_v.at[0]], sem).wait()

    @pl.loop(0, _CPT // _GRP)
    def _(grp):
        ib = base + grp * _GRP
        pltpu.sync_copy(src_hbm.at[pl.ds(ib, _GRP)], src_v)
        pltpu.sync_copy(dst_hbm.at[pl.ds(ib, _GRP)], dst_v)

        pltpu.async_copy(hw_hbm.at[src_v.at[0]], buf_a, semg_a)
        pltpu.async_copy(hw_hbm.at[src_v.at[1]], buf_b, semg_b)

        @pl.loop(0, _GRP, step=2)
        def _(g):
            _wait_gather(buf_a, semg_a)
            pltpu.async_copy(buf_a, acc.at[dst_v.at[g]], sems_a, add=True)
            _wait_gather(buf_b, semg_b)
            pltpu.async_copy(buf_b, acc.at[dst_v.at[g + 1]], sems_b, add=True)

            _wait_scatter(buf_a, sems_a)

            @pl.when(g + 2 < _GRP)
            def _():
                pltpu.async_copy(hw_hbm.at[src_v.at[g + 2]], buf_a, semg_a)

            _wait_scatter(buf_b, sems_b)

            @pl.when(g + 3 < _GRP)
            def _():
                pltpu.async_copy(hw_hbm.at[src_v.at[g + 3]], buf_b, semg_b)

    plsc.subcore_barrier()
    pltpu.sync_copy(acc.at[pl.ds(row0, _RPT)], out_hbm.at[c, pl.ds(row0, _RPT)])


_sc_aggregate = pl.kernel(
    _agg_body,
    out_type=jax.ShapeDtypeStruct((_NC, _NPAD, _DIM), jnp.float32),
    mesh=_sc_mesh,
    scratch_types=[
        pltpu.VMEM((_GRP, _CH), jnp.int32),
        pltpu.VMEM((_GRP, _CH), jnp.int32),
        pltpu.VMEM((_CH, _DIM), jnp.float32),
        pltpu.VMEM((_CH, _DIM), jnp.float32),
        pltpu.VMEM_SHARED((_NPAD, _DIM), jnp.float32),
        pltpu.SemaphoreType.DMA,
        pltpu.SemaphoreType.DMA,
        pltpu.SemaphoreType.DMA,
        pltpu.SemaphoreType.DMA,
    ],
)


def _proj_kernel(x_ref, w_ref, b_ref, o_ref):
    y = jax.lax.dot_general(
        x_ref[...], w_ref[...], (((1,), (0,)), ((), ())),
        preferred_element_type=jnp.float32,
        precision=jax.lax.Precision.DEFAULT,
    ) + b_ref[...]
    o_ref[...] = jnp.where(y >= 0, y, 0.01 * y)


def _proj(x, W, b2):
    blk = 2000
    return pl.pallas_call(
        _proj_kernel,
        grid=(_N // blk,),
        in_specs=[
            pl.BlockSpec((blk, _DIM), lambda i: (i, 0)),
            pl.BlockSpec((_DIM, _DIM), lambda i: (0, 0)),
            pl.BlockSpec((1, _DIM), lambda i: (0, 0)),
        ],
        out_specs=pl.BlockSpec((blk, _DIM), lambda i: (i, 0)),
        out_shape=jax.ShapeDtypeStruct((_N, _DIM), jnp.float32),
    )(x, W, b2)


def _gram_sigmoid_kernel(r_i_ref, r_j_ref, o_ref):
    acc = jax.lax.dot_general(
        r_i_ref[...], r_j_ref[...], (((1,), (1,)), ((), ())),
        preferred_element_type=jnp.float32,
        precision=jax.lax.Precision.DEFAULT,
    )
    o_ref[...] = jax.nn.sigmoid(acc)


def _gram_sigmoid(recons):
    blk = 200
    return pl.pallas_call(
        _gram_sigmoid_kernel,
        grid=(_N // blk,),
        in_specs=[
            pl.BlockSpec((blk, _DIM), lambda i: (i, 0)),
            pl.BlockSpec((_N, _DIM), lambda i: (0, 0)),
        ],
        out_specs=pl.BlockSpec((blk, _N), lambda i: (i, 0)),
        out_shape=jax.ShapeDtypeStruct((_N, _N), jnp.float32),
    )(recons, recons)


def kernel(x, edge_index, W, b, lam):
    src = edge_index[0]
    dst = edge_index[1]
    pad = _EPAD - _E

    src_g = jnp.concatenate([src, jnp.zeros((pad,), jnp.int32)])
    src_g = src_g.reshape(_NW * _CPT, _CH)
    src_d = jnp.concatenate([src, jnp.full((pad,), _N, jnp.int32)])
    src_d = src_d.reshape(_NW * _CPT, _CH)
    dst_p = jnp.concatenate([dst, jnp.full((pad,), _N, jnp.int32)])
    dst_p = dst_p.reshape(_NW * _CPT, _CH)

    zeros128 = jnp.zeros((_NPAD, _DIM), jnp.float32)
    ones128 = jnp.ones((_NPAD, _DIM), jnp.float32)

    degp = _sc_aggregate(ones128, src_g, src_d, zeros128)
    deg = degp[0, :, 0] + degp[1, :, 0] + 1.0
    d_invsqrt = lax.rsqrt(jnp.clip(deg, 1.0, None))[:, None]

    h = _proj(x, W, b.reshape(1, _DIM))
    h = jnp.pad(h, ((0, _NPAD - _N), (0, 0)))

    s = jax.nn.sigmoid(lam)
    sw = [s[0], s[0], s[1], s[2], s[3]]
    coef = [jnp.float32(0.0)] * 21
    for i in range(5):
        for k in range(5):
            coef[4 * i + k] = coef[4 * i + k] + sw[i] * _THETAS[i][k]

    feat = h
    recons = coef[0] * feat
    for j in range(1, 21):
        hw = feat * d_invsqrt
        parts = _sc_aggregate(hw, src_g, dst_p, zeros128)
        feat = feat - (parts[0] + parts[1] + hw) * d_invsqrt
        recons = recons + coef[j] * feat

    return _gram_sigmoid(recons[:_N])


# speedup vs baseline: 1.0297x; 1.0297x over previous
import jax
import jax.numpy as jnp
from jax import lax
from jax.experimental import pallas as pl
from jax.experimental.pallas import tpu as pltpu
from jax.experimental.pallas import tpu_sc as plsc

_THETAS = [
    [5.0, -10.0, 7.5, -2.5, 0.3125],
    [0.0, 10.0, -15.0, 7.5, -1.25],
    [0.0, 0.0, 7.5, -7.5, 1.875],
    [0.0, 0.0, 0.0, 2.5, -1.25],
    [0.0, 0.0, 0.0, 0.0, 0.3125],
]

_N = 10000
_DIM = 128
_E = 320000
_NC = 2
_NS = 16
_NW = _NC * _NS
_CH = 128
_CPT = 80
_EPAD = _NW * _CPT * _CH
_NPAD = 10112
_RPT = _NPAD // _NS

_sc_mesh = plsc.VectorSubcoreMesh(
    core_axis_name="c", subcore_axis_name="s", num_cores=_NC, num_subcores=_NS
)


_GRP = 16


def _agg_body(hw_hbm, src_hbm, dst_hbm, zero_hbm, out_hbm,
              src_v, dst_v, buf_a, buf_b, acc, sem_a, sem_b):
    c = lax.axis_index("c")
    s = lax.axis_index("s")
    w = s * _NC + c
    base = w * _CPT
    row0 = s * _RPT

    pltpu.sync_copy(zero_hbm.at[pl.ds(row0, _RPT)], acc.at[pl.ds(row0, _RPT)])
    plsc.subcore_barrier()

    @pl.loop(0, _CPT // _GRP)
    def _(grp):
        ib = base + grp * _GRP
        pltpu.sync_copy(src_hbm.at[pl.ds(ib, _GRP)], src_v)
        pltpu.sync_copy(dst_hbm.at[pl.ds(ib, _GRP)], dst_v)

        pltpu.async_copy(hw_hbm.at[src_v.at[0]], buf_a, sem_a)
        pltpu.async_copy(hw_hbm.at[src_v.at[1]], buf_b, sem_b)

        @pl.loop(0, _GRP, step=2)
        def _(g):
            pltpu.make_async_copy(hw_hbm.at[src_v.at[0]], buf_a, sem_a).wait()
            pltpu.sync_copy(buf_a, acc.at[dst_v.at[g]], add=True)

            @pl.when(g + 2 < _GRP)
            def _():
                pltpu.async_copy(hw_hbm.at[src_v.at[g + 2]], buf_a, sem_a)

            pltpu.make_async_copy(hw_hbm.at[src_v.at[1]], buf_b, sem_b).wait()
            pltpu.sync_copy(buf_b, acc.at[dst_v.at[g + 1]], add=True)

            @pl.when(g + 3 < _GRP)
            def _():
                pltpu.async_copy(hw_hbm.at[src_v.at[g + 3]], buf_b, sem_b)

    plsc.subcore_barrier()
    pltpu.sync_copy(acc.at[pl.ds(row0, _RPT)], out_hbm.at[c, pl.ds(row0, _RPT)])


_sc_aggregate = pl.kernel(
    _agg_body,
    out_type=jax.ShapeDtypeStruct((_NC, _NPAD, _DIM), jnp.float32),
    mesh=_sc_mesh,
    scratch_types=[
        pltpu.VMEM((_GRP, _CH), jnp.int32),
        pltpu.VMEM((_GRP, _CH), jnp.int32),
        pltpu.VMEM((_CH, _DIM), jnp.float32),
        pltpu.VMEM((_CH, _DIM), jnp.float32),
        pltpu.VMEM_SHARED((_NPAD, _DIM), jnp.float32),
        pltpu.SemaphoreType.DMA,
        pltpu.SemaphoreType.DMA,
    ],
)


def _proj_kernel(x_ref, w_ref, b_ref, o_ref):
    y = jax.lax.dot_general(
        x_ref[...], w_ref[...], (((1,), (0,)), ((), ())),
        preferred_element_type=jnp.float32,
        precision=jax.lax.Precision.DEFAULT,
    ) + b_ref[...]
    o_ref[...] = jnp.where(y >= 0, y, 0.01 * y)


def _proj(x, W, b2):
    blk = 2000
    return pl.pallas_call(
        _proj_kernel,
        grid=(_N // blk,),
        in_specs=[
            pl.BlockSpec((blk, _DIM), lambda i: (i, 0)),
            pl.BlockSpec((_DIM, _DIM), lambda i: (0, 0)),
            pl.BlockSpec((1, _DIM), lambda i: (0, 0)),
        ],
        out_specs=pl.BlockSpec((blk, _DIM), lambda i: (i, 0)),
        out_shape=jax.ShapeDtypeStruct((_N, _DIM), jnp.float32),
    )(x, W, b2)


def _gram_sigmoid_kernel(r_i_ref, r_j_ref, o_ref):
    acc = jax.lax.dot_general(
        r_i_ref[...], r_j_ref[...], (((1,), (1,)), ((), ())),
        preferred_element_type=jnp.float32,
        precision=jax.lax.Precision.DEFAULT,
    )
    o_ref[...] = jax.nn.sigmoid(acc)


def _gram_sigmoid(recons):
    blk = 200
    return pl.pallas_call(
        _gram_sigmoid_kernel,
        grid=(_N // blk,),
        in_specs=[
            pl.BlockSpec((blk, _DIM), lambda i: (i, 0)),
            pl.BlockSpec((_N, _DIM), lambda i: (0, 0)),
        ],
        out_specs=pl.BlockSpec((blk, _N), lambda i: (i, 0)),
        out_shape=jax.ShapeDtypeStruct((_N, _N), jnp.float32),
    )(recons, recons)


def kernel(x, edge_index, W, b, lam):
    src = edge_index[0]
    dst = edge_index[1]
    pad = _EPAD - _E

    src_g = jnp.concatenate([src, jnp.zeros((pad,), jnp.int32)])
    src_g = src_g.reshape(_NW * _CPT, _CH)
    src_d = jnp.concatenate([src, jnp.full((pad,), _N, jnp.int32)])
    src_d = src_d.reshape(_NW * _CPT, _CH)
    dst_p = jnp.concatenate([dst, jnp.full((pad,), _N, jnp.int32)])
    dst_p = dst_p.reshape(_NW * _CPT, _CH)

    zeros128 = jnp.zeros((_NPAD, _DIM), jnp.float32)
    ones128 = jnp.ones((_NPAD, _DIM), jnp.float32)

    degp = _sc_aggregate(ones128, src_g, src_d, zeros128)
    deg = degp[0, :, 0] + degp[1, :, 0] + 1.0
    d_invsqrt = lax.rsqrt(jnp.clip(deg, 1.0, None))[:, None]

    h = _proj(x, W, b.reshape(1, _DIM))
    h = jnp.pad(h, ((0, _NPAD - _N), (0, 0)))

    s = jax.nn.sigmoid(lam)
    sw = [s[0], s[0], s[1], s[2], s[3]]
    coef = [jnp.float32(0.0)] * 21
    for i in range(5):
        for k in range(5):
            coef[4 * i + k] = coef[4 * i + k] + sw[i] * _THETAS[i][k]

    feat = h
    recons = coef[0] * feat
    for j in range(1, 21):
        hw = feat * d_invsqrt
        parts = _sc_aggregate(hw, src_g, dst_p, zeros128)
        feat = feat - (parts[0] + parts[1] + hw) * d_invsqrt
        recons = recons + coef[j] * feat

    return _gram_sigmoid(recons[:_N])


# speedup vs baseline: 1.2957x; 1.2584x over previous
import jax
import jax.numpy as jnp
from jax import lax
from jax.experimental import pallas as pl
from jax.experimental.pallas import tpu as pltpu
from jax.experimental.pallas import tpu_sc as plsc

_THETAS = [
    [5.0, -10.0, 7.5, -2.5, 0.3125],
    [0.0, 10.0, -15.0, 7.5, -1.25],
    [0.0, 0.0, 7.5, -7.5, 1.875],
    [0.0, 0.0, 0.0, 2.5, -1.25],
    [0.0, 0.0, 0.0, 0.0, 0.3125],
]

_N = 10000
_DIM = 128
_E = 320000
_NC = 2
_NS = 16
_NW = _NC * _NS
_CH = 128
_CPT = 80
_EPAD = _NW * _CPT * _CH
_NPAD = 10112
_RPT = _NPAD // _NS

_sc_mesh = plsc.VectorSubcoreMesh(
    core_axis_name="c", subcore_axis_name="s", num_cores=_NC, num_subcores=_NS
)


_GRP = 16


def _agg_body(hw_hbm, src_hbm, dst_hbm, zero_hbm, out_hbm,
              src_v, dst_v, buf_a, buf_b, acc, sem_a, sem_b):
    c = lax.axis_index("c")
    s = lax.axis_index("s")
    w = s * _NC + c
    base = w * _CPT
    row0 = s * _RPT

    pltpu.sync_copy(zero_hbm.at[pl.ds(row0, _RPT)], acc.at[pl.ds(row0, _RPT)])
    plsc.subcore_barrier()

    @pl.loop(0, _CPT // _GRP)
    def _(grp):
        ib = base + grp * _GRP
        pltpu.sync_copy(src_hbm.at[pl.ds(ib, _GRP)], src_v)
        pltpu.sync_copy(dst_hbm.at[pl.ds(ib, _GRP)], dst_v)

        pltpu.async_copy(hw_hbm.at[src_v.at[0]], buf_a, sem_a)
        pltpu.async_copy(hw_hbm.at[src_v.at[1]], buf_b, sem_b)

        @pl.loop(0, _GRP, step=2)
        def _(g):
            pltpu.make_async_copy(hw_hbm.at[src_v.at[0]], buf_a, sem_a).wait()
            pltpu.sync_copy(buf_a, acc.at[dst_v.at[g]], add=True)

            @pl.when(g + 2 < _GRP)
            def _():
                pltpu.async_copy(hw_hbm.at[src_v.at[g + 2]], buf_a, sem_a)

            pltpu.make_async_copy(hw_hbm.at[src_v.at[1]], buf_b, sem_b).wait()
            pltpu.sync_copy(buf_b, acc.at[dst_v.at[g + 1]], add=True)

            @pl.when(g + 3 < _GRP)
            def _():
                pltpu.async_copy(hw_hbm.at[src_v.at[g + 3]], buf_b, sem_b)

    plsc.subcore_barrier()
    pltpu.sync_copy(acc.at[pl.ds(row0, _RPT)], out_hbm.at[c, pl.ds(row0, _RPT)])


_sc_aggregate = pl.kernel(
    _agg_body,
    out_type=jax.ShapeDtypeStruct((_NC, _NPAD, _DIM), jnp.float32),
    mesh=_sc_mesh,
    scratch_types=[
        pltpu.VMEM((_GRP, _CH), jnp.int32),
        pltpu.VMEM((_GRP, _CH), jnp.int32),
        pltpu.VMEM((_CH, _DIM), jnp.float32),
        pltpu.VMEM((_CH, _DIM), jnp.float32),
        pltpu.VMEM_SHARED((_NPAD, _DIM), jnp.float32),
        pltpu.SemaphoreType.DMA,
        pltpu.SemaphoreType.DMA,
    ],
)


def _deg_body(src_hbm, ones_hbm, zero_hbm, out_hbm, idx_v, ones_v, acc):
    c = lax.axis_index("c")
    s = lax.axis_index("s")
    w = s * _NC + c
    base = w * _CPT
    row0 = s * _RPT

    pltpu.sync_copy(ones_hbm, ones_v)
    pltpu.sync_copy(zero_hbm.at[pl.ds(row0, _RPT)], acc.at[pl.ds(row0, _RPT)])
    plsc.subcore_barrier()

    @pl.loop(0, _CPT // _GRP)
    def _(grp):
        ib = base + grp * _GRP
        pltpu.sync_copy(src_hbm.at[pl.ds(ib, _GRP)], idx_v)

        @pl.loop(0, _GRP)
        def _(g):
            pltpu.sync_copy(ones_v, acc.at[idx_v.at[g]], add=True)

    plsc.subcore_barrier()
    pltpu.sync_copy(acc.at[pl.ds(row0, _RPT)], out_hbm.at[c, pl.ds(row0, _RPT)])


_sc_degree = pl.kernel(
    _deg_body,
    out_type=jax.ShapeDtypeStruct((_NC, _NPAD, _DIM), jnp.float32),
    mesh=_sc_mesh,
    scratch_types=[
        pltpu.VMEM((_GRP, _CH), jnp.int32),
        pltpu.VMEM((_CH, _DIM), jnp.float32),
        pltpu.VMEM_SHARED((_NPAD, _DIM), jnp.float32),
    ],
)


def _proj_kernel(x_ref, w_ref, b_ref, o_ref):
    y = jax.lax.dot_general(
        x_ref[...], w_ref[...], (((1,), (0,)), ((), ())),
        preferred_element_type=jnp.float32,
        precision=jax.lax.Precision.DEFAULT,
    ) + b_ref[...]
    o_ref[...] = jnp.where(y >= 0, y, 0.01 * y)


def _proj(x, W, b2):
    blk = 2000
    return pl.pallas_call(
        _proj_kernel,
        grid=(_N // blk,),
        in_specs=[
            pl.BlockSpec((blk, _DIM), lambda i: (i, 0)),
            pl.BlockSpec((_DIM, _DIM), lambda i: (0, 0)),
            pl.BlockSpec((1, _DIM), lambda i: (0, 0)),
        ],
        out_specs=pl.BlockSpec((blk, _DIM), lambda i: (i, 0)),
        out_shape=jax.ShapeDtypeStruct((_N, _DIM), jnp.float32),
    )(x, W, b2)


def _gram_sigmoid_kernel(r_i_ref, r_j_ref, o_ref):
    acc = jax.lax.dot_general(
        r_i_ref[...], r_j_ref[...], (((1,), (1,)), ((), ())),
        preferred_element_type=jnp.float32,
        precision=jax.lax.Precision.DEFAULT,
    )
    o_ref[...] = jax.nn.sigmoid(acc)


def _gram_sigmoid(recons):
    blk = 200
    return pl.pallas_call(
        _gram_sigmoid_kernel,
        grid=(_N // blk,),
        in_specs=[
            pl.BlockSpec((blk, _DIM), lambda i: (i, 0)),
            pl.BlockSpec((_N, _DIM), lambda i: (0, 0)),
        ],
        out_specs=pl.BlockSpec((blk, _N), lambda i: (i, 0)),
        out_shape=jax.ShapeDtypeStruct((_N, _N), jnp.float32),
    )(recons, recons)


def kernel(x, edge_index, W, b, lam):
    src = edge_index[0]
    dst = edge_index[1]
    pad = _EPAD - _E

    src_g = jnp.concatenate([src, jnp.zeros((pad,), jnp.int32)])
    src_g = src_g.reshape(_NW * _CPT, _CH)
    src_d = jnp.concatenate([src, jnp.full((pad,), _N, jnp.int32)])
    src_d = src_d.reshape(_NW * _CPT, _CH)
    dst_p = jnp.concatenate([dst, jnp.full((pad,), _N, jnp.int32)])
    dst_p = dst_p.reshape(_NW * _CPT, _CH)

    zeros128 = jnp.zeros((_NPAD, _DIM), jnp.float32)
    ones128 = jnp.ones((_CH, _DIM), jnp.float32)

    degp = _sc_degree(src_d, ones128, zeros128)
    deg = degp[0, :, 0] + degp[1, :, 0] + 1.0
    d_invsqrt = lax.rsqrt(jnp.clip(deg, 1.0, None))[:, None]

    h = _proj(x, W, b.reshape(1, _DIM))
    h = jnp.pad(h, ((0, _NPAD - _N), (0, 0)))

    s = jax.nn.sigmoid(lam)
    sw = [s[0], s[0], s[1], s[2], s[3]]
    coef = [jnp.float32(0.0)] * 21
    for i in range(5):
        for k in range(5):
            coef[4 * i + k] = coef[4 * i + k] + sw[i] * _THETAS[i][k]

    feat = h
    recons = coef[0] * feat
    for j in range(1, 21):
        hw = feat * d_invsqrt
        parts = _sc_aggregate(hw, src_g, dst_p, zeros128)
        feat = feat - (parts[0] + parts[1] + hw) * d_invsqrt
        recons = recons + coef[j] * feat

    return _gram_sigmoid(recons[:_N])


# speedup vs baseline: 1.2975x; 1.0013x over previous
import jax
import jax.numpy as jnp
from jax import lax
from jax.experimental import pallas as pl
from jax.experimental.pallas import tpu as pltpu
from jax.experimental.pallas import tpu_sc as plsc

_THETAS = [
    [5.0, -10.0, 7.5, -2.5, 0.3125],
    [0.0, 10.0, -15.0, 7.5, -1.25],
    [0.0, 0.0, 7.5, -7.5, 1.875],
    [0.0, 0.0, 0.0, 2.5, -1.25],
    [0.0, 0.0, 0.0, 0.0, 0.3125],
]

_N = 10000
_DIM = 128
_E = 320000
_NC = 2
_NS = 16
_NW = _NC * _NS
_CH = 128
_CPT = 80
_EPAD = _NW * _CPT * _CH
_NPAD = 10112
_RPT = _NPAD // _NS

_sc_mesh = plsc.VectorSubcoreMesh(
    core_axis_name="c", subcore_axis_name="s", num_cores=_NC, num_subcores=_NS
)


_GRP = 16


def _agg_body(hw_hbm, src_hbm, dst_hbm, zero_hbm, out_hbm,
              src_v, dst_v, buf_a, buf_b, acc, sem_a, sem_b):
    c = lax.axis_index("c")
    s = lax.axis_index("s")
    w = s * _NC + c
    base = w * _CPT
    row0 = s * _RPT

    pltpu.sync_copy(zero_hbm.at[pl.ds(row0, _RPT)], acc.at[pl.ds(row0, _RPT)])
    plsc.subcore_barrier()

    @pl.loop(0, _CPT // _GRP)
    def _(grp):
        ib = base + grp * _GRP
        pltpu.sync_copy(src_hbm.at[pl.ds(ib, _GRP)], src_v)
        pltpu.sync_copy(dst_hbm.at[pl.ds(ib, _GRP)], dst_v)

        pltpu.async_copy(hw_hbm.at[src_v.at[0]], buf_a, sem_a)
        pltpu.async_copy(hw_hbm.at[src_v.at[1]], buf_b, sem_b)

        @pl.loop(0, _GRP, step=2)
        def _(g):
            pltpu.make_async_copy(hw_hbm.at[src_v.at[0]], buf_a, sem_a).wait()
            pltpu.sync_copy(buf_a, acc.at[dst_v.at[g]], add=True)

            @pl.when(g + 2 < _GRP)
            def _():
                pltpu.async_copy(hw_hbm.at[src_v.at[g + 2]], buf_a, sem_a)

            pltpu.make_async_copy(hw_hbm.at[src_v.at[1]], buf_b, sem_b).wait()
            pltpu.sync_copy(buf_b, acc.at[dst_v.at[g + 1]], add=True)

            @pl.when(g + 3 < _GRP)
            def _():
                pltpu.async_copy(hw_hbm.at[src_v.at[g + 3]], buf_b, sem_b)

    plsc.subcore_barrier()
    pltpu.sync_copy(acc.at[pl.ds(row0, _RPT)], out_hbm.at[c, pl.ds(row0, _RPT)])


_sc_aggregate = pl.kernel(
    _agg_body,
    out_type=jax.ShapeDtypeStruct((_NC, _NPAD, _DIM), jnp.float32),
    mesh=_sc_mesh,
    scratch_types=[
        pltpu.VMEM((_GRP, _CH), jnp.int32),
        pltpu.VMEM((_GRP, _CH), jnp.int32),
        pltpu.VMEM((_CH, _DIM), jnp.float32),
        pltpu.VMEM((_CH, _DIM), jnp.float32),
        pltpu.VMEM_SHARED((_NPAD, _DIM), jnp.float32),
        pltpu.SemaphoreType.DMA,
        pltpu.SemaphoreType.DMA,
    ],
)


def _deg_body(src_hbm, ones_hbm, zero_hbm, out_hbm, idx_v, ones_v, acc):
    c = lax.axis_index("c")
    s = lax.axis_index("s")
    w = s * _NC + c
    base = w * _CPT
    row0 = s * _RPT

    pltpu.sync_copy(ones_hbm, ones_v)
    pltpu.sync_copy(zero_hbm.at[pl.ds(row0, _RPT)], acc.at[pl.ds(row0, _RPT)])
    plsc.subcore_barrier()

    @pl.loop(0, _CPT // _GRP)
    def _(grp):
        ib = base + grp * _GRP
        pltpu.sync_copy(src_hbm.at[pl.ds(ib, _GRP)], idx_v)

        @pl.loop(0, _GRP)
        def _(g):
            pltpu.sync_copy(ones_v, acc.at[idx_v.at[g]], add=True)

    plsc.subcore_barrier()
    pltpu.sync_copy(acc.at[pl.ds(row0, _RPT)], out_hbm.at[c, pl.ds(row0, _RPT)])


_sc_degree = pl.kernel(
    _deg_body,
    out_type=jax.ShapeDtypeStruct((_NC, _NPAD, _DIM), jnp.float32),
    mesh=_sc_mesh,
    scratch_types=[
        pltpu.VMEM((_GRP, _CH), jnp.int32),
        pltpu.VMEM((_CH, _DIM), jnp.float32),
        pltpu.VMEM_SHARED((_NPAD, _DIM), jnp.float32),
    ],
)


def _proj_kernel(x_ref, w_ref, b_ref, o_ref):
    y = jax.lax.dot_general(
        x_ref[...], w_ref[...], (((1,), (0,)), ((), ())),
        preferred_element_type=jnp.float32,
        precision=jax.lax.Precision.DEFAULT,
    ) + b_ref[...]
    o_ref[...] = jnp.where(y >= 0, y, 0.01 * y)


def _proj(x, W, b2):
    blk = 2000
    return pl.pallas_call(
        _proj_kernel,
        grid=(_N // blk,),
        in_specs=[
            pl.BlockSpec((blk, _DIM), lambda i: (i, 0)),
            pl.BlockSpec((_DIM, _DIM), lambda i: (0, 0)),
            pl.BlockSpec((1, _DIM), lambda i: (0, 0)),
        ],
        out_specs=pl.BlockSpec((blk, _DIM), lambda i: (i, 0)),
        out_shape=jax.ShapeDtypeStruct((_N, _DIM), jnp.float32),
    )(x, W, b2)


def _gram_sigmoid_kernel(r_i_ref, r_j_ref, o_ref):
    acc = jax.lax.dot_general(
        r_i_ref[...], r_j_ref[...], (((1,), (1,)), ((), ())),
        preferred_element_type=jnp.float32,
        precision=jax.lax.Precision.DEFAULT,
    )
    o_ref[...] = jax.nn.sigmoid(acc)


def _gram_sigmoid(recons):
    blk = 400
    return pl.pallas_call(
        _gram_sigmoid_kernel,
        grid=(_N // blk,),
        in_specs=[
            pl.BlockSpec((blk, _DIM), lambda i: (i, 0)),
            pl.BlockSpec((_N, _DIM), lambda i: (0, 0)),
        ],
        out_specs=pl.BlockSpec((blk, _N), lambda i: (i, 0)),
        out_shape=jax.ShapeDtypeStruct((_N, _N), jnp.float32),
    )(recons, recons)


def kernel(x, edge_index, W, b, lam):
    src = edge_index[0]
    dst = edge_index[1]
    pad = _EPAD - _E

    src_g = jnp.concatenate([src, jnp.zeros((pad,), jnp.int32)])
    src_g = src_g.reshape(_NW * _CPT, _CH)
    src_d = jnp.concatenate([src, jnp.full((pad,), _N, jnp.int32)])
    src_d = src_d.reshape(_NW * _CPT, _CH)
    dst_p = jnp.concatenate([dst, jnp.full((pad,), _N, jnp.int32)])
    dst_p = dst_p.reshape(_NW * _CPT, _CH)

    zeros128 = jnp.zeros((_NPAD, _DIM), jnp.float32)
    ones128 = jnp.ones((_CH, _DIM), jnp.float32)

    degp = _sc_degree(src_d, ones128, zeros128)
    deg = degp[0, :, 0] + degp[1, :, 0] + 1.0
    d_invsqrt = lax.rsqrt(jnp.clip(deg, 1.0, None))[:, None]

    h = _proj(x, W, b.reshape(1, _DIM))
    h = jnp.pad(h, ((0, _NPAD - _N), (0, 0)))

    s = jax.nn.sigmoid(lam)
    sw = [s[0], s[0], s[1], s[2], s[3]]
    coef = [jnp.float32(0.0)] * 21
    for i in range(5):
        for k in range(5):
            coef[4 * i + k] = coef[4 * i + k] + sw[i] * _THETAS[i][k]

    feat = h
    recons = coef[0] * feat
    for j in range(1, 21):
        hw = feat * d_invsqrt
        parts = _sc_aggregate(hw, src_g, dst_p, zeros128)
        feat = feat - (parts[0] + parts[1] + hw) * d_invsqrt
        recons = recons + coef[j] * feat

    return _gram_sigmoid(recons[:_N])
